# baseline (speedup 1.0000x reference)
import functools

import jax
import jax.numpy as jnp
from jax import lax
from jax.experimental import pallas as pl
from jax.experimental.pallas import tpu as pltpu
from jax.experimental.pallas import tpu_sc as plsc

N = 10000
E = 320000
NC, NS = 2, 16
NW = NC * NS
B = 512
TOTCH = -(-E // B)
TOTCH = NS * ((TOTCH + NS - 1) // NS)
KPT = TOTCH // NS
TOTCHP = TOTCH
NROWS = 10240
RPT = NROWS // NS

_MESH = plsc.VectorSubcoreMesh(core_axis_name="c", subcore_axis_name="s",
                               num_cores=1)


def _make_agg(F):
    @functools.partial(
        pl.kernel,
        out_type=jax.ShapeDtypeStruct((NROWS, F), jnp.float32),
        mesh=_MESH,
        compiler_params=pltpu.CompilerParams(use_tc_tiling_on_sc=False),
        scratch_types=[
            pltpu.VMEM((KPT, B), jnp.int32),
            pltpu.VMEM((KPT, B), jnp.int32),
            pltpu.VMEM((B, F), jnp.float32),
            pltpu.VMEM_SHARED((NROWS, F), jnp.float32),
            pltpu.SemaphoreType.DMA,
        ],
    )
    def agg(rows_hbm, cols_hbm, y_hbm, zeros_hbm, out_hbm,
            idxr, idxc, gbuf, acc, sem):
        s = lax.axis_index("s")
        off = s * KPT
        pltpu.sync_copy(rows_hbm.at[pl.ds(off, KPT)], idxr)
        pltpu.sync_copy(cols_hbm.at[pl.ds(off, KPT)], idxc)
        for r0 in range(0, RPT, B):
            rn = min(B, RPT - r0)
            pltpu.sync_copy(zeros_hbm.at[pl.ds(r0, rn)], gbuf.at[pl.ds(0, rn)])
            pltpu.sync_copy(gbuf.at[pl.ds(0, rn)],
                            acc.at[pl.ds(s * RPT + r0, rn)])
        plsc.subcore_barrier()

        def body(j, carry):
            pltpu.async_copy(y_hbm.at[idxr.at[j]], gbuf, sem).wait()
            pltpu.sync_copy(gbuf, acc.at[idxc.at[j]], add=True)
            return carry

        lax.fori_loop(0, KPT, body, 0)
        plsc.subcore_barrier()
        for r0 in range(0, RPT, B):
            rn = min(B, RPT - r0)
            pltpu.sync_copy(acc.at[pl.ds(s * RPT + r0, rn)],
                            gbuf.at[pl.ds(0, rn)])
            pltpu.sync_copy(gbuf.at[pl.ds(0, rn)],
                            out_hbm.at[pl.ds(s * RPT + r0, rn)])

    return agg


_agg64 = _make_agg(64)
_agg32 = _make_agg(32)


@functools.partial(
    pl.kernel,
    out_type=jax.ShapeDtypeStruct((NROWS, 8), jnp.float32),
    mesh=_MESH,
    compiler_params=pltpu.CompilerParams(use_tc_tiling_on_sc=False),
    scratch_types=[
        pltpu.VMEM((KPT, B), jnp.int32),
        pltpu.VMEM((B, 8), jnp.float32),
        pltpu.VMEM((RPT, 8), jnp.float32),
        pltpu.VMEM_SHARED((NROWS, 8), jnp.float32),
        pltpu.SemaphoreType.DMA,
    ],
)
def _deg(cols_hbm, ones_hbm, zeros_hbm, out_hbm, idxc, ones_v, zbuf, acc, sem):
    s = lax.axis_index("s")
    pltpu.sync_copy(cols_hbm.at[pl.ds(s * KPT, KPT)], idxc)
    pltpu.sync_copy(ones_hbm, ones_v)
    pltpu.sync_copy(zeros_hbm, zbuf)
    pltpu.sync_copy(zbuf, acc.at[pl.ds(s * RPT, RPT)])
    plsc.subcore_barrier()

    def body(j, carry):
        pltpu.async_copy(ones_v, acc.at[idxc.at[j]], sem, add=True)
        return carry

    lax.fori_loop(0, KPT, body, 0)

    def drain(j, carry):
        pltpu.make_async_copy(ones_v, acc.at[idxc.at[0]], sem).wait()
        return carry

    lax.fori_loop(0, KPT, drain, 0)
    plsc.subcore_barrier()
    pltpu.sync_copy(acc.at[pl.ds(s * RPT, RPT)], zbuf)
    pltpu.sync_copy(zbuf, out_hbm.at[pl.ds(s * RPT, RPT)])


def _mm1_body(x_ref, w_ref, d0_ref, y_ref, d_ref):
    d = lax.rsqrt(d0_ref[...] + 1.0)
    xw = jnp.dot(x_ref[...], w_ref[...], preferred_element_type=jnp.float32)
    y_ref[...] = xw * d
    d_ref[...] = d


_mm1 = pl.pallas_call(
    _mm1_body,
    out_shape=(jax.ShapeDtypeStruct((N, 64), jnp.float32),
               jax.ShapeDtypeStruct((N, 1), jnp.float32)),
)


def _mm2_body(p0_ref, y1_ref, d_ref, b_ref, a_ref, w_ref, y2_ref):
    d = d_ref[...]
    t = d * (p0_ref[...] + y1_ref[...]) + b_ref[...]
    h = jnp.where(t >= 0, t, a_ref[0, 0] * t)
    y2_ref[...] = d * jnp.dot(h, w_ref[...],
                              preferred_element_type=jnp.float32)


_mm2 = pl.pallas_call(
    _mm2_body,
    out_shape=jax.ShapeDtypeStruct((N, 32), jnp.float32),
)


def _fin_body(p0_ref, y2_ref, d_ref, b_ref, a_ref, o_ref):
    t = d_ref[...] * (p0_ref[...] + y2_ref[...]) + b_ref[...]
    o_ref[...] = jnp.where(t >= 0, t, a_ref[0, 0] * t)


_fin = pl.pallas_call(
    _fin_body,
    out_shape=jax.ShapeDtypeStruct((N, 32), jnp.float32),
)


def kernel(x, edge_idx, W1, b1, W2, b2, a1, a2):
    row = edge_idx[0].astype(jnp.int32)
    col = edge_idx[1].astype(jnp.int32)
    pad = TOTCHP * B - E
    rowp = jnp.concatenate([row, jnp.zeros((pad,), jnp.int32)]).reshape(TOTCHP, B)
    colp = jnp.concatenate([col, jnp.full((pad,), N, jnp.int32)]).reshape(TOTCHP, B)
    ones8 = jnp.ones((B, 8), jnp.float32)
    z8 = jnp.zeros((RPT, 8), jnp.float32)
    z64 = jnp.zeros((RPT, 64), jnp.float32)
    z32 = jnp.zeros((RPT, 32), jnp.float32)

    degp = _deg(colp, ones8, z8)
    y1, d = _mm1(x, W1, degp[:N, 0:1])
    p1 = _agg64(rowp, colp, y1, z64)
    y2 = _mm2(p1[:N], y1, d, b1.reshape(1, 64), a1.reshape(1, 1), W2)
    p2 = _agg32(rowp, colp, y2, z32)
    return _fin(p2[:N], y2, d, b2.reshape(1, 32), a2.reshape(1, 1))


# speedup vs baseline: 1.5135x; 1.5135x over previous
import functools

import jax
import jax.numpy as jnp
from jax import lax
from jax.experimental import pallas as pl
from jax.experimental.pallas import tpu as pltpu
from jax.experimental.pallas import tpu_sc as plsc

N = 10000
E = 320000
NC, NS = 2, 16
B = 128
KF, KS = 97, 61
TOTCH = NS * (KF + KS)
TOTCHP = NS * KF + (NS - 1) * KS + KF
NROWS = 10240
RPT = NROWS // NS
RC = RPT // B

_MESH = plsc.VectorSubcoreMesh(core_axis_name="c", subcore_axis_name="s")


def _make_agg(F):
    @functools.partial(
        pl.kernel,
        out_type=jax.ShapeDtypeStruct((NC, NROWS, F), jnp.float32),
        mesh=_MESH,
        compiler_params=pltpu.CompilerParams(use_tc_tiling_on_sc=False),
        scratch_types=[
            pltpu.VMEM((KF, B), jnp.int32),
            pltpu.VMEM((KF, B), jnp.int32),
            pltpu.VMEM((B, F), jnp.float32),
            pltpu.VMEM_SHARED((NROWS, F), jnp.float32),
            pltpu.SemaphoreType.DMA,
        ],
    )
    def agg(rows_hbm, cols_hbm, y_hbm, zeros_hbm, out_hbm,
            idxr, idxc, gbuf, acc, sem):
        c = lax.axis_index("c")
        s = lax.axis_index("s")
        off = jnp.where(c == 0, s * KF, NS * KF + s * KS)
        kc = jnp.where(c == 0, KF, KS)
        pltpu.sync_copy(rows_hbm.at[pl.ds(off, KF)], idxr)
        pltpu.sync_copy(cols_hbm.at[pl.ds(off, KF)], idxc)
        pltpu.sync_copy(zeros_hbm, gbuf)
        for i in range(RC):
            pltpu.sync_copy(gbuf, acc.at[pl.ds(s * RPT + i * B, B)])
        plsc.subcore_barrier()

        def body(j, carry):
            pltpu.async_copy(y_hbm.at[idxr.at[j]], gbuf, sem).wait()
            pltpu.sync_copy(gbuf, acc.at[idxc.at[j]], add=True)
            return carry

        lax.fori_loop(0, kc, body, 0)
        plsc.subcore_barrier()
        for i in range(RC):
            pltpu.sync_copy(acc.at[pl.ds(s * RPT + i * B, B)], gbuf)
            pltpu.sync_copy(gbuf, out_hbm.at[c, pl.ds(s * RPT + i * B, B)])

    return agg


_agg64 = _make_agg(64)
_agg32 = _make_agg(32)


@functools.partial(
    pl.kernel,
    out_type=jax.ShapeDtypeStruct((NC, NROWS, 8), jnp.float32),
    mesh=_MESH,
    compiler_params=pltpu.CompilerParams(use_tc_tiling_on_sc=False),
    scratch_types=[
        pltpu.VMEM((KF, B), jnp.int32),
        pltpu.VMEM((B, 8), jnp.float32),
        pltpu.VMEM((B, 8), jnp.float32),
        pltpu.VMEM_SHARED((NROWS, 8), jnp.float32),
        pltpu.SemaphoreType.DMA,
    ],
)
def _deg(cols_hbm, ones_hbm, zeros_hbm, out_hbm, idxc, ones_v, zbuf, acc, sem):
    c = lax.axis_index("c")
    s = lax.axis_index("s")
    off = jnp.where(c == 0, s * KF, NS * KF + s * KS)
    kc = jnp.where(c == 0, KF, KS)
    pltpu.sync_copy(cols_hbm.at[pl.ds(off, KF)], idxc)
    pltpu.sync_copy(ones_hbm, ones_v)
    pltpu.sync_copy(zeros_hbm, zbuf)
    for i in range(RC):
        pltpu.sync_copy(zbuf, acc.at[pl.ds(s * RPT + i * B, B)])
    plsc.subcore_barrier()

    def body(j, carry):
        pltpu.async_copy(ones_v, acc.at[idxc.at[j]], sem, add=True)
        return carry

    lax.fori_loop(0, kc, body, 0)

    def drain(j, carry):
        pltpu.make_async_copy(ones_v, acc.at[idxc.at[0]], sem).wait()
        return carry

    lax.fori_loop(0, kc, drain, 0)
    plsc.subcore_barrier()
    for i in range(RC):
        pltpu.sync_copy(acc.at[pl.ds(s * RPT + i * B, B)], zbuf)
        pltpu.sync_copy(zbuf, out_hbm.at[c, pl.ds(s * RPT + i * B, B)])


def _mm1_body(x_ref, w_ref, d0_ref, d1_ref, y_ref, d_ref):
    d = lax.rsqrt(d0_ref[...] + d1_ref[...] + 1.0)
    xw = jnp.dot(x_ref[...], w_ref[...], preferred_element_type=jnp.float32)
    y_ref[...] = xw * d
    d_ref[...] = d


_mm1 = pl.pallas_call(
    _mm1_body,
    out_shape=(jax.ShapeDtypeStruct((N, 64), jnp.float32),
               jax.ShapeDtypeStruct((N, 1), jnp.float32)),
)


def _mm2_body(p0_ref, p1_ref, y1_ref, d_ref, b_ref, a_ref, w_ref, y2_ref):
    d = d_ref[...]
    t = d * (p0_ref[...] + p1_ref[...] + y1_ref[...]) + b_ref[...]
    h = jnp.where(t >= 0, t, a_ref[0, 0] * t)
    y2_ref[...] = d * jnp.dot(h, w_ref[...],
                              preferred_element_type=jnp.float32)


_mm2 = pl.pallas_call(
    _mm2_body,
    out_shape=jax.ShapeDtypeStruct((N, 32), jnp.float32),
)


def _fin_body(p0_ref, p1_ref, y2_ref, d_ref, b_ref, a_ref, o_ref):
    t = d_ref[...] * (p0_ref[...] + p1_ref[...] + y2_ref[...]) + b_ref[...]
    o_ref[...] = jnp.where(t >= 0, t, a_ref[0, 0] * t)


_fin = pl.pallas_call(
    _fin_body,
    out_shape=jax.ShapeDtypeStruct((N, 32), jnp.float32),
)


def kernel(x, edge_idx, W1, b1, W2, b2, a1, a2):
    row = edge_idx[0].astype(jnp.int32)
    col = edge_idx[1].astype(jnp.int32)
    pad = TOTCHP * B - E
    rowp = jnp.concatenate([row, jnp.zeros((pad,), jnp.int32)]).reshape(TOTCHP, B)
    colp = jnp.concatenate([col, jnp.full((pad,), N, jnp.int32)]).reshape(TOTCHP, B)
    ones8 = jnp.ones((B, 8), jnp.float32)
    z8 = jnp.zeros((B, 8), jnp.float32)
    z64 = jnp.zeros((B, 64), jnp.float32)
    z32 = jnp.zeros((B, 32), jnp.float32)

    degp = _deg(colp, ones8, z8)
    y1, d = _mm1(x, W1, degp[0, :N, 0:1], degp[1, :N, 0:1])
    p1 = _agg64(rowp, colp, y1, z64)
    y2 = _mm2(p1[0, :N], p1[1, :N], y1, d,
              b1.reshape(1, 64), a1.reshape(1, 1), W2)
    p2 = _agg32(rowp, colp, y2, z32)
    return _fin(p2[0, :N], p2[1, :N], y2, d,
                b2.reshape(1, 32), a2.reshape(1, 1))


# speedup vs baseline: 1.6471x; 1.0883x over previous
import functools

import jax
import jax.numpy as jnp
from jax import lax
from jax.experimental import pallas as pl
from jax.experimental.pallas import tpu as pltpu
from jax.experimental.pallas import tpu_sc as plsc

N = 10000
E = 320000
NC, NS = 2, 16
B = 128
KSUM = 158
TOTCH = NS * KSUM
TOTCHP = TOTCH + 36
NROWS = 10240
RPT = NROWS // NS
RC = RPT // B

_MESH = plsc.VectorSubcoreMesh(core_axis_name="c", subcore_axis_name="s")


def _make_agg(F, KF, KS):
    @functools.partial(
        pl.kernel,
        out_type=jax.ShapeDtypeStruct((NC, NROWS, F), jnp.float32),
        mesh=_MESH,
        compiler_params=pltpu.CompilerParams(use_tc_tiling_on_sc=False),
        scratch_types=[
            pltpu.VMEM((KF, B), jnp.int32),
            pltpu.VMEM((KF, B), jnp.int32),
            pltpu.VMEM((2, B, F), jnp.float32),
            pltpu.VMEM_SHARED((NROWS, F), jnp.float32),
            pltpu.SemaphoreType.DMA,
            pltpu.SemaphoreType.DMA,
        ],
    )
    def agg(rows_hbm, cols_hbm, y_hbm, zeros_hbm, out_hbm,
            idxr, idxc, gbuf, acc, sem0, sem1):
        c = lax.axis_index("c")
        s = lax.axis_index("s")
        off = jnp.where(c == 0, s * KF, NS * KF + s * KS)
        kc = jnp.where(c == 0, KF, KS)
        pltpu.sync_copy(rows_hbm.at[pl.ds(off, KF)], idxr)
        pltpu.sync_copy(cols_hbm.at[pl.ds(off, KF)], idxc)
        pltpu.sync_copy(zeros_hbm, gbuf.at[0])
        for i in range(RC):
            pltpu.sync_copy(gbuf.at[0], acc.at[pl.ds(s * RPT + i * B, B)])
        plsc.subcore_barrier()

        def body(q, carry):
            j = 2 * q
            g0 = pltpu.async_copy(y_hbm.at[idxr.at[j]], gbuf.at[0], sem0)
            g1 = pltpu.async_copy(y_hbm.at[idxr.at[j + 1]], gbuf.at[1], sem1)
            g0.wait()
            pltpu.sync_copy(gbuf.at[0], acc.at[idxc.at[j]], add=True)
            g1.wait()
            pltpu.sync_copy(gbuf.at[1], acc.at[idxc.at[j + 1]], add=True)
            return carry

        lax.fori_loop(0, kc // 2, body, 0)
        plsc.subcore_barrier()
        for i in range(RC):
            pltpu.sync_copy(acc.at[pl.ds(s * RPT + i * B, B)], gbuf.at[0])
            pltpu.sync_copy(gbuf.at[0],
                            out_hbm.at[c, pl.ds(s * RPT + i * B, B)])

    return agg


_agg64 = _make_agg(64, 96, 62)
_agg32 = _make_agg(32, 92, 66)


DKF, DKS = 92, 66


@functools.partial(
    pl.kernel,
    out_type=jax.ShapeDtypeStruct((NC, NROWS, 8), jnp.float32),
    mesh=_MESH,
    compiler_params=pltpu.CompilerParams(use_tc_tiling_on_sc=False),
    scratch_types=[
        pltpu.VMEM((DKF, B), jnp.int32),
        pltpu.VMEM((B, 8), jnp.float32),
        pltpu.VMEM((B, 8), jnp.float32),
        pltpu.VMEM_SHARED((NROWS, 8), jnp.float32),
        pltpu.SemaphoreType.DMA,
    ],
)
def _deg(cols_hbm, ones_hbm, zeros_hbm, out_hbm, idxc, ones_v, zbuf, acc, sem):
    c = lax.axis_index("c")
    s = lax.axis_index("s")
    off = jnp.where(c == 0, s * DKF, NS * DKF + s * DKS)
    kc = jnp.where(c == 0, DKF, DKS)
    pltpu.sync_copy(cols_hbm.at[pl.ds(off, DKF)], idxc)
    pltpu.sync_copy(ones_hbm, ones_v)
    pltpu.sync_copy(zeros_hbm, zbuf)
    for i in range(RC):
        pltpu.sync_copy(zbuf, acc.at[pl.ds(s * RPT + i * B, B)])
    plsc.subcore_barrier()

    def body(j, carry):
        pltpu.async_copy(ones_v, acc.at[idxc.at[j]], sem, add=True)
        return carry

    lax.fori_loop(0, kc, body, 0)

    def drain(j, carry):
        pltpu.make_async_copy(ones_v, acc.at[idxc.at[0]], sem).wait()
        return carry

    lax.fori_loop(0, kc, drain, 0)
    plsc.subcore_barrier()
    for i in range(RC):
        pltpu.sync_copy(acc.at[pl.ds(s * RPT + i * B, B)], zbuf)
        pltpu.sync_copy(zbuf, out_hbm.at[c, pl.ds(s * RPT + i * B, B)])


def _mm1_body(x_ref, w_ref, d0_ref, d1_ref, y_ref, d_ref):
    d = lax.rsqrt(d0_ref[...] + d1_ref[...] + 1.0)
    xw = jnp.dot(x_ref[...], w_ref[...], preferred_element_type=jnp.float32)
    y_ref[...] = xw * d
    d_ref[...] = d


_mm1 = pl.pallas_call(
    _mm1_body,
    out_shape=(jax.ShapeDtypeStruct((N, 64), jnp.float32),
               jax.ShapeDtypeStruct((N, 1), jnp.float32)),
)


def _mm2_body(p0_ref, p1_ref, y1_ref, d_ref, b_ref, a_ref, w_ref, y2_ref):
    d = d_ref[...]
    t = d * (p0_ref[...] + p1_ref[...] + y1_ref[...]) + b_ref[...]
    h = jnp.where(t >= 0, t, a_ref[0, 0] * t)
    y2_ref[...] = d * jnp.dot(h, w_ref[...],
                              preferred_element_type=jnp.float32)


_mm2 = pl.pallas_call(
    _mm2_body,
    out_shape=jax.ShapeDtypeStruct((N, 32), jnp.float32),
)


def _fin_body(p0_ref, p1_ref, y2_ref, d_ref, b_ref, a_ref, o_ref):
    t = d_ref[...] * (p0_ref[...] + p1_ref[...] + y2_ref[...]) + b_ref[...]
    o_ref[...] = jnp.where(t >= 0, t, a_ref[0, 0] * t)


_fin = pl.pallas_call(
    _fin_body,
    out_shape=jax.ShapeDtypeStruct((N, 32), jnp.float32),
)


def kernel(x, edge_idx, W1, b1, W2, b2, a1, a2):
    row = edge_idx[0].astype(jnp.int32)
    col = edge_idx[1].astype(jnp.int32)
    pad = TOTCHP * B - E
    rowp = jnp.concatenate([row, jnp.zeros((pad,), jnp.int32)]).reshape(TOTCHP, B)
    colp = jnp.concatenate([col, jnp.full((pad,), N, jnp.int32)]).reshape(TOTCHP, B)
    ones8 = jnp.ones((B, 8), jnp.float32)
    z8 = jnp.zeros((B, 8), jnp.float32)
    z64 = jnp.zeros((B, 64), jnp.float32)
    z32 = jnp.zeros((B, 32), jnp.float32)

    degp = _deg(colp, ones8, z8)
    y1, d = _mm1(x, W1, degp[0, :N, 0:1], degp[1, :N, 0:1])
    p1 = _agg64(rowp, colp, y1, z64)
    y2 = _mm2(p1[0, :N], p1[1, :N], y1, d,
              b1.reshape(1, 64), a1.reshape(1, 1), W2)
    p2 = _agg32(rowp, colp, y2, z32)
    return _fin(p2[0, :N], p2[1, :N], y2, d,
                b2.reshape(1, 32), a2.reshape(1, 1))


# speedup vs baseline: 1.6972x; 1.0304x over previous
import functools

import jax
import jax.numpy as jnp
from jax import lax
from jax.experimental import pallas as pl
from jax.experimental.pallas import tpu as pltpu
from jax.experimental.pallas import tpu_sc as plsc

N = 10000
E = 320000
NC, NS = 2, 16
B = 128
KSUM = 158
TOTCH = NS * KSUM
TOTCHP = TOTCH + 54
NROWS = 10240
RPT = NROWS // NS
RC = RPT // B

_MESH = plsc.VectorSubcoreMesh(core_axis_name="c", subcore_axis_name="s")


def _make_agg(F, KF, KS):
    @functools.partial(
        pl.kernel,
        out_type=jax.ShapeDtypeStruct((NC, NROWS, F), jnp.float32),
        mesh=_MESH,
        compiler_params=pltpu.CompilerParams(use_tc_tiling_on_sc=False),
        scratch_types=[
            pltpu.VMEM((KF, B), jnp.int32),
            pltpu.VMEM((KF, B), jnp.int32),
            pltpu.VMEM((2, B, F), jnp.float32),
            pltpu.VMEM_SHARED((NROWS, F), jnp.float32),
            pltpu.SemaphoreType.DMA,
            pltpu.SemaphoreType.DMA,
        ],
    )
    def agg(rows_hbm, cols_hbm, y_hbm, zeros_hbm, out_hbm,
            idxr, idxc, gbuf, acc, sem0, sem1):
        c = lax.axis_index("c")
        s = lax.axis_index("s")
        off = jnp.where(c == 0, s * KF, NS * KF + s * KS)
        kc = jnp.where(c == 0, KF, KS)
        pltpu.sync_copy(rows_hbm.at[pl.ds(off, KF)], idxr)
        pltpu.sync_copy(cols_hbm.at[pl.ds(off, KF)], idxc)
        pltpu.sync_copy(zeros_hbm, gbuf.at[0])
        for i in range(RC):
            pltpu.sync_copy(gbuf.at[0], acc.at[pl.ds(s * RPT + i * B, B)])
        plsc.subcore_barrier()

        def body(q, carry):
            j = 2 * q
            g0 = pltpu.async_copy(y_hbm.at[idxr.at[j]], gbuf.at[0], sem0)
            g1 = pltpu.async_copy(y_hbm.at[idxr.at[j + 1]], gbuf.at[1], sem1)
            g0.wait()
            pltpu.sync_copy(gbuf.at[0], acc.at[idxc.at[j]], add=True)
            g1.wait()
            pltpu.sync_copy(gbuf.at[1], acc.at[idxc.at[j + 1]], add=True)
            return carry

        lax.fori_loop(0, kc // 2, body, 0)
        plsc.subcore_barrier()
        for i in range(RC):
            pltpu.sync_copy(acc.at[pl.ds(s * RPT + i * B, B)], gbuf.at[0])
            pltpu.sync_copy(gbuf.at[0],
                            out_hbm.at[c, pl.ds(s * RPT + i * B, B)])

    return agg


_agg64 = _make_agg(64, 106, 52)
_agg32 = _make_agg(32, 96, 62)


DKF, DKS = 96, 62


@functools.partial(
    pl.kernel,
    out_type=jax.ShapeDtypeStruct((NC, NROWS, 8), jnp.float32),
    mesh=_MESH,
    compiler_params=pltpu.CompilerParams(use_tc_tiling_on_sc=False),
    scratch_types=[
        pltpu.VMEM((DKF, B), jnp.int32),
        pltpu.VMEM((B, 8), jnp.float32),
        pltpu.VMEM((B, 8), jnp.float32),
        pltpu.VMEM_SHARED((NROWS, 8), jnp.float32),
        pltpu.SemaphoreType.DMA,
    ],
)
def _deg(cols_hbm, ones_hbm, zeros_hbm, out_hbm, idxc, ones_v, zbuf, acc, sem):
    c = lax.axis_index("c")
    s = lax.axis_index("s")
    off = jnp.where(c == 0, s * DKF, NS * DKF + s * DKS)
    kc = jnp.where(c == 0, DKF, DKS)
    pltpu.sync_copy(cols_hbm.at[pl.ds(off, DKF)], idxc)
    pltpu.sync_copy(ones_hbm, ones_v)
    pltpu.sync_copy(zeros_hbm, zbuf)
    for i in range(RC):
        pltpu.sync_copy(zbuf, acc.at[pl.ds(s * RPT + i * B, B)])
    plsc.subcore_barrier()

    def body(j, carry):
        pltpu.async_copy(ones_v, acc.at[idxc.at[j]], sem, add=True)
        return carry

    lax.fori_loop(0, kc, body, 0)

    def drain(j, carry):
        pltpu.make_async_copy(ones_v, acc.at[idxc.at[0]], sem).wait()
        return carry

    lax.fori_loop(0, kc, drain, 0)
    plsc.subcore_barrier()
    for i in range(RC):
        pltpu.sync_copy(acc.at[pl.ds(s * RPT + i * B, B)], zbuf)
        pltpu.sync_copy(zbuf, out_hbm.at[c, pl.ds(s * RPT + i * B, B)])


def _mm1_body(x_ref, w_ref, d0_ref, d1_ref, y_ref, d_ref):
    d = lax.rsqrt(d0_ref[...] + d1_ref[...] + 1.0)
    xw = jnp.dot(x_ref[...], w_ref[...], preferred_element_type=jnp.float32)
    y_ref[...] = xw * d
    d_ref[...] = d


_mm1 = pl.pallas_call(
    _mm1_body,
    out_shape=(jax.ShapeDtypeStruct((N, 64), jnp.float32),
               jax.ShapeDtypeStruct((N, 1), jnp.float32)),
)


def _mm2_body(p0_ref, p1_ref, y1_ref, d_ref, b_ref, a_ref, w_ref, y2_ref):
    d = d_ref[...]
    t = d * (p0_ref[...] + p1_ref[...] + y1_ref[...]) + b_ref[...]
    h = jnp.where(t >= 0, t, a_ref[0, 0] * t)
    y2_ref[...] = d * jnp.dot(h, w_ref[...],
                              preferred_element_type=jnp.float32)


_mm2 = pl.pallas_call(
    _mm2_body,
    out_shape=jax.ShapeDtypeStruct((N, 32), jnp.float32),
)


def _fin_body(p0_ref, p1_ref, y2_ref, d_ref, b_ref, a_ref, o_ref):
    t = d_ref[...] * (p0_ref[...] + p1_ref[...] + y2_ref[...]) + b_ref[...]
    o_ref[...] = jnp.where(t >= 0, t, a_ref[0, 0] * t)


_fin = pl.pallas_call(
    _fin_body,
    out_shape=jax.ShapeDtypeStruct((N, 32), jnp.float32),
)


def kernel(x, edge_idx, W1, b1, W2, b2, a1, a2):
    row = edge_idx[0].astype(jnp.int32)
    col = edge_idx[1].astype(jnp.int32)
    pad = TOTCHP * B - E
    rowp = jnp.concatenate([row, jnp.zeros((pad,), jnp.int32)]).reshape(TOTCHP, B)
    colp = jnp.concatenate([col, jnp.full((pad,), N, jnp.int32)]).reshape(TOTCHP, B)
    ones8 = jnp.ones((B, 8), jnp.float32)
    z8 = jnp.zeros((B, 8), jnp.float32)
    z64 = jnp.zeros((B, 64), jnp.float32)
    z32 = jnp.zeros((B, 32), jnp.float32)

    degp = _deg(colp, ones8, z8)
    y1, d = _mm1(x, W1, degp[0, :N, 0:1], degp[1, :N, 0:1])
    p1 = _agg64(rowp, colp, y1, z64)
    y2 = _mm2(p1[0, :N], p1[1, :N], y1, d,
              b1.reshape(1, 64), a1.reshape(1, 1), W2)
    p2 = _agg32(rowp, colp, y2, z32)
    return _fin(p2[0, :N], p2[1, :N], y2, d,
                b2.reshape(1, 32), a2.reshape(1, 1))


# speedup vs baseline: 1.7191x; 1.0129x over previous
import functools

import jax
import jax.numpy as jnp
from jax import lax
from jax.experimental import pallas as pl
from jax.experimental.pallas import tpu as pltpu
from jax.experimental.pallas import tpu_sc as plsc

N = 10000
E = 320000
NC, NS = 2, 16
B = 128
KSUM = 158
TOTCH = NS * KSUM
TOTCHP = TOTCH + 62
NROWS = 10240
RPT = NROWS // NS
RC = RPT // B

_MESH = plsc.VectorSubcoreMesh(core_axis_name="c", subcore_axis_name="s")


def _make_agg(F, KF, KS):
    @functools.partial(
        pl.kernel,
        out_type=jax.ShapeDtypeStruct((NC, NROWS, F), jnp.float32),
        mesh=_MESH,
        compiler_params=pltpu.CompilerParams(use_tc_tiling_on_sc=False),
        scratch_types=[
            pltpu.VMEM((KF, B), jnp.int32),
            pltpu.VMEM((KF, B), jnp.int32),
            pltpu.VMEM((2, B, F), jnp.float32),
            pltpu.VMEM_SHARED((NROWS, F), jnp.float32),
            pltpu.SemaphoreType.DMA,
            pltpu.SemaphoreType.DMA,
        ],
    )
    def agg(rows_hbm, cols_hbm, y_hbm, zeros_hbm, out_hbm,
            idxr, idxc, gbuf, acc, sem0, sem1):
        c = lax.axis_index("c")
        s = lax.axis_index("s")
        off = jnp.where(c == 0, s * KF, NS * KF + s * KS)
        kc = jnp.where(c == 0, KF, KS)
        pltpu.sync_copy(rows_hbm.at[pl.ds(off, KF)], idxr)
        pltpu.sync_copy(cols_hbm.at[pl.ds(off, KF)], idxc)
        pltpu.sync_copy(zeros_hbm, gbuf.at[0])
        for i in range(RC):
            pltpu.sync_copy(gbuf.at[0], acc.at[pl.ds(s * RPT + i * B, B)])
        plsc.subcore_barrier()

        def body(q, carry):
            j = 2 * q
            g0 = pltpu.async_copy(y_hbm.at[idxr.at[j]], gbuf.at[0], sem0)
            g1 = pltpu.async_copy(y_hbm.at[idxr.at[j + 1]], gbuf.at[1], sem1)
            g0.wait()
            pltpu.sync_copy(gbuf.at[0], acc.at[idxc.at[j]], add=True)
            g1.wait()
            pltpu.sync_copy(gbuf.at[1], acc.at[idxc.at[j + 1]], add=True)
            return carry

        lax.fori_loop(0, kc // 2, body, 0)
        plsc.subcore_barrier()
        for i in range(RC):
            pltpu.sync_copy(acc.at[pl.ds(s * RPT + i * B, B)], gbuf.at[0])
            pltpu.sync_copy(gbuf.at[0],
                            out_hbm.at[c, pl.ds(s * RPT + i * B, B)])

    return agg


_agg64 = _make_agg(64, 110, 48)
_agg32 = _make_agg(32, 98, 60)


DKF, DKS = 96, 62


@functools.partial(
    pl.kernel,
    out_type=jax.ShapeDtypeStruct((NC, NROWS, 8), jnp.float32),
    mesh=_MESH,
    compiler_params=pltpu.CompilerParams(use_tc_tiling_on_sc=False),
    scratch_types=[
        pltpu.VMEM((DKF, B), jnp.int32),
        pltpu.VMEM((B, 8), jnp.float32),
        pltpu.VMEM((B, 8), jnp.float32),
        pltpu.VMEM_SHARED((NROWS, 8), jnp.float32),
        pltpu.SemaphoreType.DMA,
    ],
)
def _deg(cols_hbm, ones_hbm, zeros_hbm, out_hbm, idxc, ones_v, zbuf, acc, sem):
    c = lax.axis_index("c")
    s = lax.axis_index("s")
    off = jnp.where(c == 0, s * DKF, NS * DKF + s * DKS)
    kc = jnp.where(c == 0, DKF, DKS)
    pltpu.sync_copy(cols_hbm.at[pl.ds(off, DKF)], idxc)
    pltpu.sync_copy(ones_hbm, ones_v)
    pltpu.sync_copy(zeros_hbm, zbuf)
    for i in range(RC):
        pltpu.sync_copy(zbuf, acc.at[pl.ds(s * RPT + i * B, B)])
    plsc.subcore_barrier()

    def body(j, carry):
        pltpu.async_copy(ones_v, acc.at[idxc.at[j]], sem, add=True)
        return carry

    lax.fori_loop(0, kc, body, 0)

    def drain(j, carry):
        pltpu.make_async_copy(ones_v, acc.at[idxc.at[0]], sem).wait()
        return carry

    lax.fori_loop(0, kc, drain, 0)
    plsc.subcore_barrier()
    for i in range(RC):
        pltpu.sync_copy(acc.at[pl.ds(s * RPT + i * B, B)], zbuf)
        pltpu.sync_copy(zbuf, out_hbm.at[c, pl.ds(s * RPT + i * B, B)])


def _mm1_body(x_ref, w_ref, d0_ref, d1_ref, y_ref, d_ref):
    d = lax.rsqrt(d0_ref[...] + d1_ref[...] + 1.0)
    xw = jnp.dot(x_ref[...], w_ref[...], preferred_element_type=jnp.float32)
    y_ref[...] = xw * d
    d_ref[...] = d


_mm1 = pl.pallas_call(
    _mm1_body,
    out_shape=(jax.ShapeDtypeStruct((N, 64), jnp.float32),
               jax.ShapeDtypeStruct((N, 1), jnp.float32)),
)


def _mm2_body(p0_ref, p1_ref, y1_ref, d_ref, b_ref, a_ref, w_ref, y2_ref):
    d = d_ref[...]
    t = d * (p0_ref[...] + p1_ref[...] + y1_ref[...]) + b_ref[...]
    h = jnp.where(t >= 0, t, a_ref[0, 0] * t)
    y2_ref[...] = d * jnp.dot(h, w_ref[...],
                              preferred_element_type=jnp.float32)


_mm2 = pl.pallas_call(
    _mm2_body,
    out_shape=jax.ShapeDtypeStruct((N, 32), jnp.float32),
)


def _fin_body(p0_ref, p1_ref, y2_ref, d_ref, b_ref, a_ref, o_ref):
    t = d_ref[...] * (p0_ref[...] + p1_ref[...] + y2_ref[...]) + b_ref[...]
    o_ref[...] = jnp.where(t >= 0, t, a_ref[0, 0] * t)


_fin = pl.pallas_call(
    _fin_body,
    out_shape=jax.ShapeDtypeStruct((N, 32), jnp.float32),
)


def kernel(x, edge_idx, W1, b1, W2, b2, a1, a2):
    row = edge_idx[0].astype(jnp.int32)
    col = edge_idx[1].astype(jnp.int32)
    pad = TOTCHP * B - E
    rowp = jnp.concatenate([row, jnp.zeros((pad,), jnp.int32)]).reshape(TOTCHP, B)
    colp = jnp.concatenate([col, jnp.full((pad,), N, jnp.int32)]).reshape(TOTCHP, B)
    ones8 = jnp.ones((B, 8), jnp.float32)
    z8 = jnp.zeros((B, 8), jnp.float32)
    z64 = jnp.zeros((B, 64), jnp.float32)
    z32 = jnp.zeros((B, 32), jnp.float32)

    degp = _deg(colp, ones8, z8)
    y1, d = _mm1(x, W1, degp[0, :N, 0:1], degp[1, :N, 0:1])
    p1 = _agg64(rowp, colp, y1, z64)
    y2 = _mm2(p1[0, :N], p1[1, :N], y1, d,
              b1.reshape(1, 64), a1.reshape(1, 1), W2)
    p2 = _agg32(rowp, colp, y2, z32)
    return _fin(p2[0, :N], p2[1, :N], y2, d,
                b2.reshape(1, 32), a2.reshape(1, 1))
